# baseline (device time: 263400 ns/iter reference)
import jax
import jax.numpy as jnp
from jax import lax
from jax.experimental import pallas as pl
from jax.experimental.pallas import tpu as pltpu


def kernel(x, W, labels):
    T, D = x.shape
    V_loc = W.shape[1]
    BV = 2048
    n_chunks = V_loc // BV

    my_z = lax.axis_index("z")
    local_col = labels - my_z * V_loc
    valid = (local_col >= 0) & (local_col < V_loc)
    safe_col = jnp.clip(local_col, 0, V_loc - 1)
    Wg = jnp.take(W, safe_col, axis=1)
    ll_local = jnp.where(
        valid, jnp.einsum("td,dt->t", x, Wg), jnp.float32(0)
    ).astype(jnp.float32)

    def body(x_ref, W_ref, ll_ref, out_ref,
             s_ref, comm_ref, recv_ref, send_sem, recv_sem):
        step = pl.program_id(0)

        @pl.when(step == 0)
        def _init():
            s_ref[:] = jnp.zeros((T, 1), dtype=jnp.float32)

        logits = jnp.dot(x_ref[:], W_ref[:],
                         preferred_element_type=jnp.float32)
        expd = jnp.exp(logits)
        ones = jnp.ones((BV, 1), dtype=jnp.float32)
        s_ref[:] = s_ref[:] + jnp.dot(expd, ones,
                                      preferred_element_type=jnp.float32)

        @pl.when(step == n_chunks - 1)
        def _exchange():
            my_x = lax.axis_index("x")
            my_y = lax.axis_index("y")
            z_here = lax.axis_index("z")
            comm_ref[0, :] = s_ref[:, 0]
            comm_ref[1, :] = ll_ref[:]
            rdma = pltpu.make_async_remote_copy(
                src_ref=comm_ref,
                dst_ref=recv_ref,
                send_sem=send_sem,
                recv_sem=recv_sem,
                device_id=(my_x, my_y, 1 - z_here),
                device_id_type=pl.DeviceIdType.MESH,
            )
            rdma.start()
            rdma.wait()

            s_tot = s_ref[:, 0] + recv_ref[0, :]
            ll_tot = ll_ref[:] + recv_ref[1, :]
            out_ref[:] = jnp.log(s_tot) - ll_tot

    return pl.pallas_call(
        body,
        grid=(n_chunks,),
        in_specs=[
            pl.BlockSpec(memory_space=pltpu.VMEM),
            pl.BlockSpec((D, BV), lambda i: (0, i)),
            pl.BlockSpec(memory_space=pltpu.VMEM),
        ],
        out_specs=pl.BlockSpec(memory_space=pltpu.VMEM),
        out_shape=jax.ShapeDtypeStruct((T,), jnp.float32),
        scratch_shapes=[
            pltpu.VMEM((T, 1), jnp.float32),
            pltpu.VMEM((2, T), jnp.float32),
            pltpu.VMEM((2, T), jnp.float32),
            pltpu.SemaphoreType.DMA,
            pltpu.SemaphoreType.DMA,
        ],
        compiler_params=pltpu.CompilerParams(
            dimension_semantics=("arbitrary",),
            vmem_limit_bytes=100 * 1024 * 1024,
        ),
    )(x, W, ll_local)


# device time: 47620 ns/iter; 5.5313x vs baseline; 5.5313x over previous
import jax
import jax.numpy as jnp
from jax import lax
from jax.experimental import pallas as pl
from jax.experimental.pallas import tpu as pltpu


def kernel(x, W, labels):
    T, D = x.shape
    V_loc = W.shape[1]
    BV = 2048
    Q = 4
    V_q = V_loc // Q
    n_chunks = V_q // BV

    my_q = (2 * lax.axis_index("x") + lax.axis_index("y")).astype(jnp.int32)

    def body(q_ref, x_ref, W_ref, labels_ref, out_ref,
             buf, s_ref, ll_ref, comm_ref, recv_ref,
             send_sems, recv_sems):
        k = pl.program_id(0)

        @pl.when(k == 0)
        def _init():
            s_ref[:] = jnp.zeros((T,), dtype=jnp.float32)
            ll_ref[:] = jnp.zeros((T,), dtype=jnp.float32)

        buf[:] = jnp.dot(x_ref[:], W_ref[:],
                         preferred_element_type=jnp.float32)
        lg = buf[:]
        s_ref[:] = s_ref[:] + jnp.sum(jnp.exp(lg), axis=1)

        my_z = lax.axis_index("z")
        off = my_z * V_loc + (q_ref[0] * n_chunks + k) * BV
        cols = lax.broadcasted_iota(jnp.int32, (T, BV), 1)
        hit = cols == (labels_ref[:] - off)[:, None]
        ll_ref[:] = ll_ref[:] + jnp.sum(
            jnp.where(hit, lg, jnp.float32(0)), axis=1)

        @pl.when(k == n_chunks - 1)
        def _allreduce():
            my_x = lax.axis_index("x")
            my_y = lax.axis_index("y")
            z_here = lax.axis_index("z")
            partners = [
                (1 - my_x, my_y, z_here),
                (my_x, 1 - my_y, z_here),
                (my_x, my_y, 1 - z_here),
            ]
            for i, partner in enumerate(partners):
                comm_ref[i, 0, :] = s_ref[:]
                comm_ref[i, 1, :] = ll_ref[:]
                rdma = pltpu.make_async_remote_copy(
                    src_ref=comm_ref.at[i],
                    dst_ref=recv_ref.at[i],
                    send_sem=send_sems.at[i],
                    recv_sem=recv_sems.at[i],
                    device_id=partner,
                    device_id_type=pl.DeviceIdType.MESH,
                )
                rdma.start()
                rdma.wait()
                s_ref[:] = s_ref[:] + recv_ref[i, 0, :]
                ll_ref[:] = ll_ref[:] + recv_ref[i, 1, :]

            out_ref[:] = jnp.log(s_ref[:]) - ll_ref[:]

    grid_spec = pltpu.PrefetchScalarGridSpec(
        num_scalar_prefetch=1,
        grid=(n_chunks,),
        in_specs=[
            pl.BlockSpec(memory_space=pltpu.VMEM),
            pl.BlockSpec((D, BV), lambda i, q: (0, q[0] * n_chunks + i)),
            pl.BlockSpec(memory_space=pltpu.VMEM),
        ],
        out_specs=pl.BlockSpec(memory_space=pltpu.VMEM),
        scratch_shapes=[
            pltpu.VMEM((T, BV), jnp.float32),
            pltpu.VMEM((T,), jnp.float32),
            pltpu.VMEM((T,), jnp.float32),
            pltpu.VMEM((3, 2, T), jnp.float32),
            pltpu.VMEM((3, 2, T), jnp.float32),
            pltpu.SemaphoreType.DMA((3,)),
            pltpu.SemaphoreType.DMA((3,)),
        ],
    )

    return pl.pallas_call(
        body,
        grid_spec=grid_spec,
        out_shape=jax.ShapeDtypeStruct((T,), jnp.float32),
        compiler_params=pltpu.CompilerParams(
            dimension_semantics=("arbitrary",),
            vmem_limit_bytes=100 * 1024 * 1024,
        ),
    )(jnp.reshape(my_q, (1,)), x, W, labels)
